# Initial kernel scaffold; baseline (speedup 1.0000x reference)
import jax
import jax.numpy as jnp
from jax.experimental import pallas as pl


def kernel(x, edge_index, W_init, W_ih, W_hh, b_ih, b_hh):
    raise NotImplementedError("write your pallas kernel here")


# speedup vs baseline: 14.2713x; 14.2713x over previous
import functools

import jax
import jax.numpy as jnp
from jax import lax
from jax.experimental import pallas as pl
from jax.experimental.pallas import tpu as pltpu
from jax.experimental.pallas import tpu_sc as plsc

_N = 10000
_D = 128
_DH = _D // 2
_NC = 2
_NS = 16
_NW = _NC * _NS
_CHUNK = 128
_NPAD = 10112
_RPT = _NPAD // _NS


def _sc_mesh():
    return plsc.VectorSubcoreMesh(core_axis_name="c", subcore_axis_name="s")


def _make_sc_deg(epad):
    cpt = epad // (_CHUNK * _NW)

    @functools.partial(
        pl.kernel,
        out_type=jax.ShapeDtypeStruct((_NC, _NPAD, 16), jnp.float32),
        mesh=_sc_mesh(),
        scratch_types=[
            pltpu.VMEM((cpt, _CHUNK), jnp.int32),
            pltpu.VMEM((_CHUNK, 16), jnp.float32),
            pltpu.VMEM((_RPT, 16), jnp.float32),
            pltpu.VMEM_SHARED((_NPAD, 16), jnp.float32),
        ],
        compiler_params=pltpu.CompilerParams(use_tc_tiling_on_sc=False),
    )
    def deg_kernel(dst_hbm, deg_out, dstv, ones_v, zero_v, acc_sh):
        cid = lax.axis_index("c")
        sid = lax.axis_index("s")
        wid = sid * _NC + cid

        def fill_ones(i, _):
            ones_v[i] = jnp.full((16,), 1.0, jnp.float32)
            return 0

        lax.fori_loop(0, _CHUNK, fill_ones, 0)

        def fill_zero(i, _):
            zero_v[i] = jnp.zeros((16,), jnp.float32)
            return 0

        lax.fori_loop(0, _RPT, fill_zero, 0)
        pltpu.sync_copy(zero_v, acc_sh.at[pl.ds(sid * _RPT, _RPT)])
        plsc.subcore_barrier()

        pltpu.sync_copy(dst_hbm.at[pl.ds(wid * cpt, cpt)], dstv)

        def chunk(k, _):
            pltpu.sync_copy(ones_v, acc_sh.at[dstv.at[k]], add=True)
            return 0

        lax.fori_loop(0, cpt, chunk, 0)
        plsc.subcore_barrier()
        pltpu.sync_copy(
            acc_sh.at[pl.ds(sid * _RPT, _RPT)],
            deg_out.at[cid, pl.ds(sid * _RPT, _RPT)],
        )

    return deg_kernel


def _make_sc_msg(epad):
    cpt = epad // (_CHUNK * _NS)

    @functools.partial(
        pl.kernel,
        out_type=jax.ShapeDtypeStruct((_NC, _NPAD, _DH), jnp.float32),
        mesh=_sc_mesh(),
        scratch_types=[
            pltpu.VMEM((cpt, _CHUNK), jnp.int32),
            pltpu.VMEM((cpt, _CHUNK), jnp.int32),
            pltpu.VMEM((_CHUNK, _DH), jnp.float32),
            pltpu.VMEM((_CHUNK, _DH), jnp.float32),
            pltpu.VMEM_SHARED((_NPAD, _DH), jnp.float32),
            pltpu.SemaphoreType.DMA,
            pltpu.SemaphoreType.DMA,
        ],
        compiler_params=pltpu.CompilerParams(use_tc_tiling_on_sc=False),
    )
    def msg_kernel(y_hbm, srcg_hbm, dst_hbm, acc_out, srcv, dstv,
                   rows_a, rows_b, acc_sh, sem_a, sem_b):
        cid = lax.axis_index("c")
        sid = lax.axis_index("s")

        def fill_zero(i, _):
            for j in range(_DH // 16):
                rows_a[i, pl.ds(j * 16, 16)] = jnp.zeros((16,), jnp.float32)
            return 0

        lax.fori_loop(0, _CHUNK, fill_zero, 0)
        base = sid * _RPT
        nfull, rem = _RPT // _CHUNK, _RPT % _CHUNK
        for r in range(nfull):
            pltpu.sync_copy(rows_a, acc_sh.at[pl.ds(base + r * _CHUNK, _CHUNK)])
        if rem:
            pltpu.sync_copy(rows_a.at[pl.ds(0, rem)],
                            acc_sh.at[pl.ds(base + nfull * _CHUNK, rem)])
        plsc.subcore_barrier()

        pltpu.sync_copy(srcg_hbm.at[cid, pl.ds(sid * cpt, cpt)], srcv)
        pltpu.sync_copy(dst_hbm.at[pl.ds(sid * cpt, cpt)], dstv)

        def chunk(k, _):
            pltpu.async_copy(y_hbm.at[srcv.at[k]], rows_a, sem_a).wait()
            pltpu.sync_copy(rows_a, acc_sh.at[dstv.at[k]], add=True)
            return 0

        lax.fori_loop(0, cpt, chunk, 0)
        plsc.subcore_barrier()
        pltpu.sync_copy(
            acc_sh.at[pl.ds(base, _RPT)],
            acc_out.at[cid, pl.ds(base, _RPT)],
        )

    return msg_kernel


def _tc_y_body(x_ref, deg_ref, wi_ref, wih_ref, whh_ref, bih_ref, bhh_ref,
               y_ref):
    w0 = wi_ref[...]
    gi = lax.dot_general(w0, wih_ref[...], (((1,), (1,)), ((), ())),
                         preferred_element_type=jnp.float32) + bih_ref[...]
    gh = lax.dot_general(w0, whh_ref[...], (((1,), (1,)), ((), ())),
                         preferred_element_type=jnp.float32) + bhh_ref[...]
    i_r, i_z, i_n = gi[:, :_D], gi[:, _D:2 * _D], gi[:, 2 * _D:]
    h_r, h_z, h_n = gh[:, :_D], gh[:, _D:2 * _D], gh[:, 2 * _D:]
    r = jax.nn.sigmoid(i_r + h_r)
    z = jax.nn.sigmoid(i_z + h_z)
    n = jnp.tanh(i_n + r * h_n)
    w = (1.0 - z) * n + z * w0

    deg = deg_ref[0] + deg_ref[1] + 1.0
    dinv = lax.rsqrt(deg)[:, 0:1]
    xw = jnp.dot(x_ref[...], w, preferred_element_type=jnp.float32)
    y_ref[...] = xw * dinv


def _tc_fin_body(acc_ref, y_ref, deg_ref, out_ref):
    deg = deg_ref[0] + deg_ref[1] + 1.0
    dinv = lax.rsqrt(deg)[:, 0:1]
    acc = jnp.concatenate([acc_ref[0], acc_ref[1]], axis=1)
    out_ref[...] = ((acc + y_ref[...]) * dinv)[:_N]


def kernel(x, edge_index, W_init, W_ih, W_hh, b_ih, b_hh):
    e = edge_index.shape[1]
    ealign = _CHUNK * _NS * 8
    epad = ((e + ealign - 1) // ealign) * ealign

    src = edge_index[0]
    dst = edge_index[1]
    padlen = epad - e
    pad_ix = jnp.full((padlen,), _N, jnp.int32)
    src_p = jnp.concatenate([src, pad_ix])
    dst2d = jnp.concatenate([dst, pad_ix]).reshape(epad // _CHUNK, _CHUNK)
    srcg = jnp.stack([2 * src_p, 2 * src_p + 1]).reshape(
        _NC, epad // _CHUNK, _CHUNK)
    x_pad = jnp.pad(x, ((0, _NPAD - _N), (0, 0)))

    deg16 = _make_sc_deg(epad)(dst2d)

    y = pl.pallas_call(
        _tc_y_body,
        out_shape=jax.ShapeDtypeStruct((_NPAD, _D), jnp.float32),
    )(x_pad, deg16, W_init, W_ih, W_hh, b_ih.reshape(1, 3 * _D),
      b_hh.reshape(1, 3 * _D))

    acc = _make_sc_msg(epad)(y.reshape(2 * _NPAD, _DH), srcg, dst2d)

    out = pl.pallas_call(
        _tc_fin_body,
        out_shape=jax.ShapeDtypeStruct((_N, _D), jnp.float32),
    )(acc, y, deg16)
    return out


# speedup vs baseline: 16.6611x; 1.1675x over previous
import functools

import jax
import jax.numpy as jnp
from jax import lax
from jax.experimental import pallas as pl
from jax.experimental.pallas import tpu as pltpu
from jax.experimental.pallas import tpu_sc as plsc

_N = 10000
_D = 128
_DH = _D // 2
_NC = 2
_NS = 16
_NW = _NC * _NS
_CHUNK = 128
_NPAD = 10112
_NBUF = 4
_LEAD = 2
_RPT = _NPAD // _NS


def _sc_mesh():
    return plsc.VectorSubcoreMesh(core_axis_name="c", subcore_axis_name="s")


def _make_sc_deg(epad):
    cpt = epad // (_CHUNK * _NW)

    @functools.partial(
        pl.kernel,
        out_type=jax.ShapeDtypeStruct((_NC, _NPAD, 16), jnp.float32),
        mesh=_sc_mesh(),
        scratch_types=[
            pltpu.VMEM((cpt, _CHUNK), jnp.int32),
            pltpu.VMEM((_CHUNK, 16), jnp.float32),
            pltpu.VMEM((_RPT, 16), jnp.float32),
            pltpu.VMEM_SHARED((_NPAD, 16), jnp.float32),
        ],
        compiler_params=pltpu.CompilerParams(use_tc_tiling_on_sc=False),
    )
    def deg_kernel(dst_hbm, deg_out, dstv, ones_v, zero_v, acc_sh):
        cid = lax.axis_index("c")
        sid = lax.axis_index("s")
        wid = sid * _NC + cid

        def fill_ones(i, _):
            ones_v[i] = jnp.full((16,), 1.0, jnp.float32)
            return 0

        lax.fori_loop(0, _CHUNK, fill_ones, 0)

        def fill_zero(i, _):
            zero_v[i] = jnp.zeros((16,), jnp.float32)
            return 0

        lax.fori_loop(0, _RPT, fill_zero, 0)
        pltpu.sync_copy(zero_v, acc_sh.at[pl.ds(sid * _RPT, _RPT)])
        plsc.subcore_barrier()

        pltpu.sync_copy(dst_hbm.at[pl.ds(wid * cpt, cpt)], dstv)

        def chunk(k, _):
            pltpu.sync_copy(ones_v, acc_sh.at[dstv.at[k]], add=True)
            return 0

        lax.fori_loop(0, cpt, chunk, 0)
        plsc.subcore_barrier()
        pltpu.sync_copy(
            acc_sh.at[pl.ds(sid * _RPT, _RPT)],
            deg_out.at[cid, pl.ds(sid * _RPT, _RPT)],
        )

    return deg_kernel


def _make_sc_msg(epad):
    cpt = epad // (_CHUNK * _NS)

    @functools.partial(
        pl.kernel,
        out_type=jax.ShapeDtypeStruct((_NC, _NPAD, _DH), jnp.float32),
        mesh=_sc_mesh(),
        scratch_types=[
            pltpu.VMEM((cpt, _CHUNK), jnp.int32),
            pltpu.VMEM((cpt, _CHUNK), jnp.int32),
            [pltpu.VMEM((_CHUNK, _DH), jnp.float32) for _ in range(_NBUF)],
            pltpu.VMEM_SHARED((_NPAD, _DH), jnp.float32),
            [pltpu.SemaphoreType.DMA for _ in range(_NBUF)],
            [pltpu.SemaphoreType.DMA for _ in range(_NBUF)],
        ],
        compiler_params=pltpu.CompilerParams(use_tc_tiling_on_sc=False),
    )
    def msg_kernel(y_hbm, srcg_hbm, dst_hbm, acc_out, srcv, dstv,
                   rows, acc_sh, gsem, ssem):
        cid = lax.axis_index("c")
        sid = lax.axis_index("s")

        def fill_zero(i, _):
            for j in range(_DH // 16):
                rows[0][i, pl.ds(j * 16, 16)] = jnp.zeros((16,), jnp.float32)
            return 0

        lax.fori_loop(0, _CHUNK, fill_zero, 0)
        base = sid * _RPT
        nfull, rem = _RPT // _CHUNK, _RPT % _CHUNK
        for r in range(nfull):
            pltpu.sync_copy(rows[0],
                            acc_sh.at[pl.ds(base + r * _CHUNK, _CHUNK)])
        if rem:
            pltpu.sync_copy(rows[0].at[pl.ds(0, rem)],
                            acc_sh.at[pl.ds(base + nfull * _CHUNK, rem)])
        plsc.subcore_barrier()

        pltpu.sync_copy(srcg_hbm.at[cid, pl.ds(sid * cpt, cpt)], srcv)
        pltpu.sync_copy(dst_hbm.at[pl.ds(sid * cpt, cpt)], dstv)

        for b in range(_LEAD):
            pltpu.async_copy(y_hbm.at[srcv.at[b]], rows[b], gsem[b])

        def visit(k, b):
            kf = k + _LEAD - _NBUF

            @pl.when(kf >= 0)
            def _():
                bf = (b + _LEAD) % _NBUF
                pltpu.make_async_copy(
                    rows[bf], acc_sh.at[dstv.at[jnp.maximum(kf, 0)]],
                    ssem[bf]).wait()

            @pl.when(k + _LEAD < cpt)
            def _():
                bg = (b + _LEAD) % _NBUF
                pltpu.async_copy(y_hbm.at[srcv.at[k + _LEAD]], rows[bg],
                                 gsem[bg])

            pltpu.make_async_copy(y_hbm.at[srcv.at[k]], rows[b], gsem[b]).wait()
            pltpu.async_copy(rows[b], acc_sh.at[dstv.at[k]], ssem[b],
                             add=True)

        ngroups = cpt // _NBUF

        def group(g, _):
            k0 = g * _NBUF
            for b in range(_NBUF):
                visit(k0 + b, b)
            return 0

        lax.fori_loop(0, ngroups, group, 0)
        for k in range(cpt - (_NBUF - _LEAD), cpt):
            b = k % _NBUF
            pltpu.make_async_copy(rows[b], acc_sh.at[dstv.at[k]],
                                  ssem[b]).wait()
        plsc.subcore_barrier()
        pltpu.sync_copy(
            acc_sh.at[pl.ds(base, _RPT)],
            acc_out.at[cid, pl.ds(base, _RPT)],
        )

    return msg_kernel


def _tc_y_body(x_ref, deg_ref, wi_ref, wih_ref, whh_ref, bih_ref, bhh_ref,
               y_ref):
    w0 = wi_ref[...]
    gi = lax.dot_general(w0, wih_ref[...], (((1,), (1,)), ((), ())),
                         preferred_element_type=jnp.float32) + bih_ref[...]
    gh = lax.dot_general(w0, whh_ref[...], (((1,), (1,)), ((), ())),
                         preferred_element_type=jnp.float32) + bhh_ref[...]
    i_r, i_z, i_n = gi[:, :_D], gi[:, _D:2 * _D], gi[:, 2 * _D:]
    h_r, h_z, h_n = gh[:, :_D], gh[:, _D:2 * _D], gh[:, 2 * _D:]
    r = jax.nn.sigmoid(i_r + h_r)
    z = jax.nn.sigmoid(i_z + h_z)
    n = jnp.tanh(i_n + r * h_n)
    w = (1.0 - z) * n + z * w0

    deg = deg_ref[0] + deg_ref[1] + 1.0
    dinv = lax.rsqrt(deg)[:, 0:1]
    xw = jnp.dot(x_ref[...], w, preferred_element_type=jnp.float32)
    y_ref[...] = xw * dinv


def _tc_fin_body(acc_ref, y_ref, deg_ref, out_ref):
    deg = deg_ref[0] + deg_ref[1] + 1.0
    dinv = lax.rsqrt(deg)[:, 0:1]
    acc = jnp.concatenate([acc_ref[0], acc_ref[1]], axis=1)
    out_ref[...] = ((acc + y_ref[...]) * dinv)[:_N]


def kernel(x, edge_index, W_init, W_ih, W_hh, b_ih, b_hh):
    e = edge_index.shape[1]
    ealign = _CHUNK * _NS * 8
    epad = ((e + ealign - 1) // ealign) * ealign

    src = edge_index[0]
    dst = edge_index[1]
    padlen = epad - e
    pad_ix = jnp.full((padlen,), _N, jnp.int32)
    src_p = jnp.concatenate([src, pad_ix])
    dst2d = jnp.concatenate([dst, pad_ix]).reshape(epad // _CHUNK, _CHUNK)
    srcg = jnp.stack([2 * src_p, 2 * src_p + 1]).reshape(
        _NC, epad // _CHUNK, _CHUNK)
    x_pad = jnp.pad(x, ((0, _NPAD - _N), (0, 0)))

    deg16 = _make_sc_deg(epad)(dst2d)

    y = pl.pallas_call(
        _tc_y_body,
        out_shape=jax.ShapeDtypeStruct((_NPAD, _D), jnp.float32),
    )(x_pad, deg16, W_init, W_ih, W_hh, b_ih.reshape(1, 3 * _D),
      b_hh.reshape(1, 3 * _D))

    acc = _make_sc_msg(epad)(y.reshape(2 * _NPAD, _DH), srcg, dst2d)

    out = pl.pallas_call(
        _tc_fin_body,
        out_shape=jax.ShapeDtypeStruct((_N, _D), jnp.float32),
    )(acc, y, deg16)
    return out


# speedup vs baseline: 16.7884x; 1.0076x over previous
import functools

import jax
import jax.numpy as jnp
from jax import lax
from jax.experimental import pallas as pl
from jax.experimental.pallas import tpu as pltpu
from jax.experimental.pallas import tpu_sc as plsc

_N = 10000
_D = 128
_DH = _D // 2
_NC = 2
_NS = 16
_NW = _NC * _NS
_CHUNK = 128
_NPAD = 10112
_NBUF = 5
_LEAD = 3
_RPT = _NPAD // _NS


def _sc_mesh():
    return plsc.VectorSubcoreMesh(core_axis_name="c", subcore_axis_name="s")


def _make_sc_deg(epad):
    cpt = epad // (_CHUNK * _NW)

    @functools.partial(
        pl.kernel,
        out_type=jax.ShapeDtypeStruct((_NC, _NPAD, 16), jnp.float32),
        mesh=_sc_mesh(),
        scratch_types=[
            pltpu.VMEM((cpt, _CHUNK), jnp.int32),
            pltpu.VMEM((_CHUNK, 16), jnp.float32),
            pltpu.VMEM((_RPT, 16), jnp.float32),
            pltpu.VMEM_SHARED((_NPAD, 16), jnp.float32),
        ],
        compiler_params=pltpu.CompilerParams(use_tc_tiling_on_sc=False),
    )
    def deg_kernel(dst_hbm, deg_out, dstv, ones_v, zero_v, acc_sh):
        cid = lax.axis_index("c")
        sid = lax.axis_index("s")
        wid = sid * _NC + cid

        def fill_ones(i, _):
            ones_v[i] = jnp.full((16,), 1.0, jnp.float32)
            return 0

        lax.fori_loop(0, _CHUNK, fill_ones, 0)

        def fill_zero(i, _):
            zero_v[i] = jnp.zeros((16,), jnp.float32)
            return 0

        lax.fori_loop(0, _RPT, fill_zero, 0)
        pltpu.sync_copy(zero_v, acc_sh.at[pl.ds(sid * _RPT, _RPT)])
        plsc.subcore_barrier()

        pltpu.sync_copy(dst_hbm.at[pl.ds(wid * cpt, cpt)], dstv)

        def chunk(k, _):
            pltpu.sync_copy(ones_v, acc_sh.at[dstv.at[k]], add=True)
            return 0

        lax.fori_loop(0, cpt, chunk, 0)
        plsc.subcore_barrier()
        pltpu.sync_copy(
            acc_sh.at[pl.ds(sid * _RPT, _RPT)],
            deg_out.at[cid, pl.ds(sid * _RPT, _RPT)],
        )

    return deg_kernel


def _make_sc_msg(epad):
    cpt = epad // (_CHUNK * _NS)

    @functools.partial(
        pl.kernel,
        out_type=jax.ShapeDtypeStruct((_NC, _NPAD, _DH), jnp.float32),
        mesh=_sc_mesh(),
        scratch_types=[
            pltpu.VMEM((cpt, _CHUNK), jnp.int32),
            pltpu.VMEM((cpt, _CHUNK), jnp.int32),
            [pltpu.VMEM((_CHUNK, _DH), jnp.float32) for _ in range(_NBUF)],
            pltpu.VMEM_SHARED((_NPAD, _DH), jnp.float32),
            [pltpu.SemaphoreType.DMA for _ in range(_NBUF)],
            [pltpu.SemaphoreType.DMA for _ in range(_NBUF)],
        ],
        compiler_params=pltpu.CompilerParams(use_tc_tiling_on_sc=False),
    )
    def msg_kernel(y_hbm, srcg_hbm, dst_hbm, acc_out, srcv, dstv,
                   rows, acc_sh, gsem, ssem):
        cid = lax.axis_index("c")
        sid = lax.axis_index("s")

        def fill_zero(i, _):
            for j in range(_DH // 16):
                rows[0][i, pl.ds(j * 16, 16)] = jnp.zeros((16,), jnp.float32)
            return 0

        lax.fori_loop(0, _CHUNK, fill_zero, 0)
        base = sid * _RPT
        nfull, rem = _RPT // _CHUNK, _RPT % _CHUNK
        for r in range(nfull):
            pltpu.sync_copy(rows[0],
                            acc_sh.at[pl.ds(base + r * _CHUNK, _CHUNK)])
        if rem:
            pltpu.sync_copy(rows[0].at[pl.ds(0, rem)],
                            acc_sh.at[pl.ds(base + nfull * _CHUNK, rem)])
        plsc.subcore_barrier()

        pltpu.sync_copy(srcg_hbm.at[cid, pl.ds(sid * cpt, cpt)], srcv)
        pltpu.sync_copy(dst_hbm.at[pl.ds(sid * cpt, cpt)], dstv)

        for b in range(_LEAD):
            pltpu.async_copy(y_hbm.at[srcv.at[b]], rows[b], gsem[b])

        def visit(k, b):
            kf = k + _LEAD - _NBUF

            @pl.when(kf >= 0)
            def _():
                bf = (b + _LEAD) % _NBUF
                pltpu.make_async_copy(
                    rows[bf], acc_sh.at[dstv.at[jnp.maximum(kf, 0)]],
                    ssem[bf]).wait()

            @pl.when(k + _LEAD < cpt)
            def _():
                bg = (b + _LEAD) % _NBUF
                pltpu.async_copy(y_hbm.at[srcv.at[k + _LEAD]], rows[bg],
                                 gsem[bg])

            pltpu.make_async_copy(y_hbm.at[srcv.at[k]], rows[b], gsem[b]).wait()
            pltpu.async_copy(rows[b], acc_sh.at[dstv.at[k]], ssem[b],
                             add=True)

        ngroups = cpt // _NBUF

        def group(g, _):
            k0 = g * _NBUF
            for b in range(_NBUF):
                visit(k0 + b, b)
            return 0

        lax.fori_loop(0, ngroups, group, 0)
        for k in range(cpt - (_NBUF - _LEAD), cpt):
            b = k % _NBUF
            pltpu.make_async_copy(rows[b], acc_sh.at[dstv.at[k]],
                                  ssem[b]).wait()
        plsc.subcore_barrier()
        pltpu.sync_copy(
            acc_sh.at[pl.ds(base, _RPT)],
            acc_out.at[cid, pl.ds(base, _RPT)],
        )

    return msg_kernel


def _tc_xw_body(x_ref, wi_ref, wih_ref, whh_ref, bih_ref, bhh_ref, xw_ref):
    w0 = wi_ref[...]
    gi = lax.dot_general(w0, wih_ref[...], (((1,), (1,)), ((), ())),
                         preferred_element_type=jnp.float32) + bih_ref[...]
    gh = lax.dot_general(w0, whh_ref[...], (((1,), (1,)), ((), ())),
                         preferred_element_type=jnp.float32) + bhh_ref[...]
    i_r, i_z, i_n = gi[:, :_D], gi[:, _D:2 * _D], gi[:, 2 * _D:]
    h_r, h_z, h_n = gh[:, :_D], gh[:, _D:2 * _D], gh[:, 2 * _D:]
    r = jax.nn.sigmoid(i_r + h_r)
    z = jax.nn.sigmoid(i_z + h_z)
    n = jnp.tanh(i_n + r * h_n)
    w = (1.0 - z) * n + z * w0
    xw_ref[...] = jnp.dot(x_ref[...], w, preferred_element_type=jnp.float32)


def _tc_scale_body(xw_ref, deg_ref, y_ref):
    deg = deg_ref[0] + deg_ref[1] + 1.0
    dinv = lax.rsqrt(deg)[:, 0:1]
    y_ref[...] = xw_ref[...] * dinv


def _tc_fin_body(acc_ref, y_ref, deg_ref, out_ref):
    deg = deg_ref[0] + deg_ref[1] + 1.0
    dinv = lax.rsqrt(deg)[:, 0:1]
    acc = jnp.concatenate([acc_ref[0], acc_ref[1]], axis=1)
    out_ref[...] = ((acc + y_ref[...]) * dinv)[:_N]


def kernel(x, edge_index, W_init, W_ih, W_hh, b_ih, b_hh):
    e = edge_index.shape[1]
    ealign = _CHUNK * _NS * 8
    epad = ((e + ealign - 1) // ealign) * ealign

    src = edge_index[0]
    dst = edge_index[1]
    padlen = epad - e
    pad_ix = jnp.full((padlen,), _N, jnp.int32)
    src_p = jnp.concatenate([src, pad_ix])
    dst2d = jnp.concatenate([dst, pad_ix]).reshape(epad // _CHUNK, _CHUNK)
    srcg = jnp.stack([2 * src_p, 2 * src_p + 1]).reshape(
        _NC, epad // _CHUNK, _CHUNK)
    x_pad = jnp.pad(x, ((0, _NPAD - _N), (0, 0)))

    deg16 = _make_sc_deg(epad)(dst2d)

    xw = pl.pallas_call(
        _tc_xw_body,
        out_shape=jax.ShapeDtypeStruct((_NPAD, _D), jnp.float32),
    )(x_pad, W_init, W_ih, W_hh, b_ih.reshape(1, 3 * _D),
      b_hh.reshape(1, 3 * _D))

    y = pl.pallas_call(
        _tc_scale_body,
        out_shape=jax.ShapeDtypeStruct((_NPAD, _D), jnp.float32),
    )(xw, deg16)

    acc = _make_sc_msg(epad)(y.reshape(2 * _NPAD, _DH), srcg, dst2d)

    out = pl.pallas_call(
        _tc_fin_body,
        out_shape=jax.ShapeDtypeStruct((_N, _D), jnp.float32),
    )(acc, y, deg16)
    return out


# speedup vs baseline: 16.8381x; 1.0030x over previous
import functools

import jax
import jax.numpy as jnp
from jax import lax
from jax.experimental import pallas as pl
from jax.experimental.pallas import tpu as pltpu
from jax.experimental.pallas import tpu_sc as plsc

_N = 10000
_D = 128
_DH = _D // 2
_NC = 2
_NS = 16
_NW = _NC * _NS
_CHUNK = 128
_NPAD = 10112
_NBUF = 5
_LEAD = 4
_RPT = _NPAD // _NS


def _sc_mesh():
    return plsc.VectorSubcoreMesh(core_axis_name="c", subcore_axis_name="s")


def _make_sc_deg(epad):
    cpt = epad // (_CHUNK * _NW)

    @functools.partial(
        pl.kernel,
        out_type=jax.ShapeDtypeStruct((_NC, _NPAD, 16), jnp.float32),
        mesh=_sc_mesh(),
        scratch_types=[
            pltpu.VMEM((cpt, _CHUNK), jnp.int32),
            pltpu.VMEM((_CHUNK, 16), jnp.float32),
            pltpu.VMEM((_RPT, 16), jnp.float32),
            pltpu.VMEM_SHARED((_NPAD, 16), jnp.float32),
        ],
        compiler_params=pltpu.CompilerParams(use_tc_tiling_on_sc=False),
    )
    def deg_kernel(dst_hbm, deg_out, dstv, ones_v, zero_v, acc_sh):
        cid = lax.axis_index("c")
        sid = lax.axis_index("s")
        wid = sid * _NC + cid

        def fill_ones(i, _):
            ones_v[i] = jnp.full((16,), 1.0, jnp.float32)
            return 0

        lax.fori_loop(0, _CHUNK, fill_ones, 0)

        def fill_zero(i, _):
            zero_v[i] = jnp.zeros((16,), jnp.float32)
            return 0

        lax.fori_loop(0, _RPT, fill_zero, 0)
        pltpu.sync_copy(zero_v, acc_sh.at[pl.ds(sid * _RPT, _RPT)])
        plsc.subcore_barrier()

        pltpu.sync_copy(dst_hbm.at[pl.ds(wid * cpt, cpt)], dstv)

        def chunk(k, _):
            pltpu.sync_copy(ones_v, acc_sh.at[dstv.at[k]], add=True)
            return 0

        lax.fori_loop(0, cpt, chunk, 0)
        plsc.subcore_barrier()
        pltpu.sync_copy(
            acc_sh.at[pl.ds(sid * _RPT, _RPT)],
            deg_out.at[cid, pl.ds(sid * _RPT, _RPT)],
        )

    return deg_kernel


def _make_sc_msg(epad):
    cpt = epad // (_CHUNK * _NS)

    @functools.partial(
        pl.kernel,
        out_type=jax.ShapeDtypeStruct((_NC, _NPAD, _DH), jnp.float32),
        mesh=_sc_mesh(),
        scratch_types=[
            pltpu.VMEM((cpt, _CHUNK), jnp.int32),
            pltpu.VMEM((cpt, _CHUNK), jnp.int32),
            [pltpu.VMEM((_CHUNK, _DH), jnp.float32) for _ in range(_NBUF)],
            pltpu.VMEM_SHARED((_NPAD, _DH), jnp.float32),
            [pltpu.SemaphoreType.DMA for _ in range(_NBUF)],
            [pltpu.SemaphoreType.DMA for _ in range(_NBUF)],
        ],
        compiler_params=pltpu.CompilerParams(use_tc_tiling_on_sc=False),
    )
    def msg_kernel(y_hbm, srcg_hbm, dst_hbm, acc_out, srcv, dstv,
                   rows, acc_sh, gsem, ssem):
        cid = lax.axis_index("c")
        sid = lax.axis_index("s")

        def fill_zero(i, _):
            for j in range(_DH // 16):
                rows[0][i, pl.ds(j * 16, 16)] = jnp.zeros((16,), jnp.float32)
            return 0

        lax.fori_loop(0, _CHUNK, fill_zero, 0)
        base = sid * _RPT
        nfull, rem = _RPT // _CHUNK, _RPT % _CHUNK
        for r in range(nfull):
            pltpu.sync_copy(rows[0],
                            acc_sh.at[pl.ds(base + r * _CHUNK, _CHUNK)])
        if rem:
            pltpu.sync_copy(rows[0].at[pl.ds(0, rem)],
                            acc_sh.at[pl.ds(base + nfull * _CHUNK, rem)])
        plsc.subcore_barrier()

        pltpu.sync_copy(srcg_hbm.at[cid, pl.ds(sid * cpt, cpt)], srcv)
        pltpu.sync_copy(dst_hbm.at[pl.ds(sid * cpt, cpt)], dstv)

        for b in range(_LEAD):
            pltpu.async_copy(y_hbm.at[srcv.at[b]], rows[b], gsem[b])

        def visit(k, b):
            kf = k + _LEAD - _NBUF

            @pl.when(kf >= 0)
            def _():
                bf = (b + _LEAD) % _NBUF
                pltpu.make_async_copy(
                    rows[bf], acc_sh.at[dstv.at[jnp.maximum(kf, 0)]],
                    ssem[bf]).wait()

            @pl.when(k + _LEAD < cpt)
            def _():
                bg = (b + _LEAD) % _NBUF
                pltpu.async_copy(y_hbm.at[srcv.at[k + _LEAD]], rows[bg],
                                 gsem[bg])

            pltpu.make_async_copy(y_hbm.at[srcv.at[k]], rows[b], gsem[b]).wait()
            pltpu.async_copy(rows[b], acc_sh.at[dstv.at[k]], ssem[b],
                             add=True)

        ngroups = cpt // _NBUF

        def group(g, _):
            k0 = g * _NBUF
            for b in range(_NBUF):
                visit(k0 + b, b)
            return 0

        lax.fori_loop(0, ngroups, group, 0)
        for k in range(cpt - (_NBUF - _LEAD), cpt):
            b = k % _NBUF
            pltpu.make_async_copy(rows[b], acc_sh.at[dstv.at[k]],
                                  ssem[b]).wait()
        plsc.subcore_barrier()
        pltpu.sync_copy(
            acc_sh.at[pl.ds(base, _RPT)],
            acc_out.at[cid, pl.ds(base, _RPT)],
        )

    return msg_kernel


def _tc_xw_body(x_ref, wi_ref, wih_ref, whh_ref, bih_ref, bhh_ref, xw_ref):
    w0 = wi_ref[...]
    gi = lax.dot_general(w0, wih_ref[...], (((1,), (1,)), ((), ())),
                         preferred_element_type=jnp.float32) + bih_ref[...]
    gh = lax.dot_general(w0, whh_ref[...], (((1,), (1,)), ((), ())),
                         preferred_element_type=jnp.float32) + bhh_ref[...]
    i_r, i_z, i_n = gi[:, :_D], gi[:, _D:2 * _D], gi[:, 2 * _D:]
    h_r, h_z, h_n = gh[:, :_D], gh[:, _D:2 * _D], gh[:, 2 * _D:]
    r = jax.nn.sigmoid(i_r + h_r)
    z = jax.nn.sigmoid(i_z + h_z)
    n = jnp.tanh(i_n + r * h_n)
    w = (1.0 - z) * n + z * w0
    xw_ref[...] = jnp.dot(x_ref[...], w, preferred_element_type=jnp.float32)


def _tc_scale_body(xw_ref, deg_ref, y_ref):
    deg = deg_ref[0] + deg_ref[1] + 1.0
    dinv = lax.rsqrt(deg)[:, 0:1]
    y_ref[...] = xw_ref[...] * dinv


def _tc_fin_body(acc_ref, y_ref, deg_ref, out_ref):
    deg = deg_ref[0] + deg_ref[1] + 1.0
    dinv = lax.rsqrt(deg)[:, 0:1]
    acc = jnp.concatenate([acc_ref[0], acc_ref[1]], axis=1)
    out_ref[...] = ((acc + y_ref[...]) * dinv)[:_N]


def kernel(x, edge_index, W_init, W_ih, W_hh, b_ih, b_hh):
    e = edge_index.shape[1]
    ealign = _CHUNK * _NS * 8
    epad = ((e + ealign - 1) // ealign) * ealign

    src = edge_index[0]
    dst = edge_index[1]
    padlen = epad - e
    pad_ix = jnp.full((padlen,), _N, jnp.int32)
    src_p = jnp.concatenate([src, pad_ix])
    dst2d = jnp.concatenate([dst, pad_ix]).reshape(epad // _CHUNK, _CHUNK)
    srcg = jnp.stack([2 * src_p, 2 * src_p + 1]).reshape(
        _NC, epad // _CHUNK, _CHUNK)
    x_pad = jnp.pad(x, ((0, _NPAD - _N), (0, 0)))

    deg16 = _make_sc_deg(epad)(dst2d)

    xw = pl.pallas_call(
        _tc_xw_body,
        out_shape=jax.ShapeDtypeStruct((_NPAD, _D), jnp.float32),
    )(x_pad, W_init, W_ih, W_hh, b_ih.reshape(1, 3 * _D),
      b_hh.reshape(1, 3 * _D))

    y = pl.pallas_call(
        _tc_scale_body,
        out_shape=jax.ShapeDtypeStruct((_NPAD, _D), jnp.float32),
    )(xw, deg16)

    acc = _make_sc_msg(epad)(y.reshape(2 * _NPAD, _DH), srcg, dst2d)

    out = pl.pallas_call(
        _tc_fin_body,
        out_shape=jax.ShapeDtypeStruct((_N, _D), jnp.float32),
    )(acc, y, deg16)
    return out
